# baseline (speedup 1.0000x reference)
import functools

import jax
import jax.numpy as jnp
from jax.experimental import pallas as pl
from jax.experimental.pallas import tpu as pltpu

D_IN = 768
H = 128
HH = 2 * H
O_REAL = 85
OP = 128
NEG_BIG = -1e30
TT = 16
BT = 256


def _round_up(x, m):
    return (x + m - 1) // m * m


def _rnn_kernel(len_ref, xa_ref, xb_ref, w1_ref, brow_ref, w2_ref,
                b1_ref, wfc_ref, bfc_ref, out_ref, xp_ref, h_ref, last_ref):
    t = pl.program_id(1)
    n_t = pl.num_programs(1)
    tbase = t * TT

    @pl.when(t == 0)
    def _init():
        h_ref[...] = jnp.zeros_like(h_ref)
        last_ref[...] = jnp.zeros_like(last_ref)

    @pl.when(tbase <= jnp.max(len_ref[...]))
    def _compute():
        bt = h_ref.shape[0]
        ht = TT // 2
        for k, xr in ((0, xa_ref), (1, xb_ref)):
            xv = xr[...].reshape(bt * ht, D_IN)
            xp_all = (jnp.dot(xv, w1_ref[...],
                              preferred_element_type=jnp.float32)
                      + brow_ref[...])
            xp3 = xp_all.reshape(bt, ht, HH)
            for s in range(ht):
                r = (k * ht + s) * bt
                xp_ref[r:r + bt, :] = xp3[:, s, :]

        len_m1 = len_ref[...] - 1
        h = h_ref[...]
        last = last_ref[...]
        w2 = w2_ref[...]
        for s in range(TT):
            p = jnp.dot(h, w2, preferred_element_type=jnp.float32)
            h = jnp.tanh(p + xp_ref[s * bt:(s + 1) * bt, :])
            g = tbase + s
            last = jnp.where(len_m1 == g - 1, h[:, H:], last)
        h_ref[...] = h
        last_ref[...] = last

    @pl.when(t == n_t - 1)
    def _final():
        len_m1 = len_ref[...] - 1
        pf = jnp.dot(h_ref[...], w2_ref[...],
                     preferred_element_type=jnp.float32)[:, H:] + b1_ref[...]
        h1f = jnp.tanh(pf)
        last = jnp.where(len_m1 == n_t * TT - 1, h1f, last_ref[...])
        logits = (jnp.dot(last, wfc_ref[...],
                          preferred_element_type=jnp.float32) + bfc_ref[...])
        m = jnp.max(logits, axis=1, keepdims=True)
        sh = logits - m
        lse = jnp.log(jnp.sum(jnp.exp(sh), axis=1, keepdims=True))
        out_ref[...] = sh - lse


@functools.partial(jax.jit, static_argnames=())
def kernel(x, lengths, wih0, whh0, b0, wih1, whh1, b1, wfc, bfc):
    B, T, D = x.shape
    B_pad = _round_up(max(B, 8), 8)
    bt = BT if B_pad % BT == 0 else B_pad
    n_b = B_pad // bt
    T_pad = _round_up(T, TT)
    n_t = T_pad // TT

    xf = x.astype(jnp.float32)
    if B_pad != B or T_pad != T:
        xf = jnp.pad(xf, ((0, B_pad - B), (0, T_pad - T), (0, 0)))

    len_pad = lengths.astype(jnp.int32)
    if B_pad != B:
        len_pad = jnp.pad(len_pad, (0, B_pad - B), constant_values=1)
    len_col = len_pad.reshape(B_pad, 1)

    w1 = jnp.pad(wih0, ((0, H), (0, 0))).T
    brow = jnp.concatenate([b0, b1]).reshape(1, HH)
    w2 = jnp.concatenate(
        [jnp.pad(whh0, ((0, 0), (0, H))),
         jnp.concatenate([wih1, whh1], axis=1)],
        axis=0).T
    b1row = b1.reshape(1, H)
    wfcp = jnp.pad(wfc, ((0, OP - O_REAL), (0, 0))).T
    bfcp = jnp.concatenate(
        [bfc, jnp.full((OP - O_REAL,), NEG_BIG, jnp.float32)]).reshape(1, OP)

    out = pl.pallas_call(
        _rnn_kernel,
        out_shape=jax.ShapeDtypeStruct((B_pad, O_REAL), jnp.float32),
        grid=(n_b, n_t),
        in_specs=[
            pl.BlockSpec((bt, 1), lambda b, t: (b, 0)),
            pl.BlockSpec((bt, TT // 2, D), lambda b, t: (b, 2 * t, 0)),
            pl.BlockSpec((bt, TT // 2, D), lambda b, t: (b, 2 * t + 1, 0)),
            pl.BlockSpec((D, HH), lambda b, t: (0, 0)),
            pl.BlockSpec((1, HH), lambda b, t: (0, 0)),
            pl.BlockSpec((HH, HH), lambda b, t: (0, 0)),
            pl.BlockSpec((1, H), lambda b, t: (0, 0)),
            pl.BlockSpec((H, OP), lambda b, t: (0, 0)),
            pl.BlockSpec((1, OP), lambda b, t: (0, 0)),
        ],
        out_specs=pl.BlockSpec((bt, OP), lambda b, t: (b, 0)),
        scratch_shapes=[
            pltpu.VMEM((TT * bt, HH), jnp.float32),
            pltpu.VMEM((bt, HH), jnp.float32),
            pltpu.VMEM((bt, H), jnp.float32),
        ],
        compiler_params=pltpu.CompilerParams(
            dimension_semantics=("parallel", "arbitrary"),
            vmem_limit_bytes=60 * 2**20),
    )(len_col, xf, xf, w1, brow, w2, b1row, wfcp, bfcp)

    return out if B_pad == B else out[:B]


# speedup vs baseline: 1.1171x; 1.1171x over previous
import functools

import jax
import jax.numpy as jnp
from jax.experimental import pallas as pl
from jax.experimental.pallas import tpu as pltpu

D_IN = 768
H = 128
HH = 2 * H
O_REAL = 85
OP = 128
NEG_BIG = -1e30
TT = 16
BT = 256


def _round_up(x, m):
    return (x + m - 1) // m * m


def _rnn_kernel(len_ref, x_ref, wih0_ref, whh0_ref, wih1_ref, whh1_ref,
                b0_ref, b1_ref, wfc_ref, bfc_ref, out_ref,
                w1_ref, w2_ref, brow_ref, wfcp_ref, bfcp_ref,
                xp_ref, h_ref, last_ref):
    t = pl.program_id(1)
    n_t = pl.num_programs(1)
    tbase = t * TT
    bt = h_ref.shape[0]

    @pl.when(t == 0)
    def _init():
        h_ref[...] = jnp.zeros_like(h_ref)
        last_ref[...] = jnp.zeros_like(last_ref)
        w1_ref[:, :H] = jnp.swapaxes(wih0_ref[...], 0, 1)
        w1_ref[:, H:] = jnp.zeros((D_IN, H), jnp.float32)
        w2_ref[:H, :H] = jnp.swapaxes(whh0_ref[...], 0, 1)
        w2_ref[:H, H:] = jnp.swapaxes(wih1_ref[...], 0, 1)
        w2_ref[H:, :H] = jnp.zeros((H, H), jnp.float32)
        w2_ref[H:, H:] = jnp.swapaxes(whh1_ref[...], 0, 1)
        brow_ref[:, :H] = b0_ref[...]
        brow_ref[:, H:] = b1_ref[...]
        wfcp_ref[...] = jnp.concatenate(
            [jnp.swapaxes(wfc_ref[...], 0, 1),
             jnp.zeros((H, OP - O_REAL), jnp.float32)], axis=1)
        bfcp_ref[...] = jnp.concatenate(
            [bfc_ref[...],
             jnp.full((1, OP - O_REAL), NEG_BIG, jnp.float32)], axis=1)

    @pl.when(tbase <= jnp.max(len_ref[...]))
    def _compute():
        xv = x_ref[...].reshape(bt * TT, D_IN)
        xp_all = (jnp.dot(xv, w1_ref[...],
                          preferred_element_type=jnp.float32)
                  + brow_ref[...])
        xp3 = xp_all.reshape(bt, TT, HH)
        for s in range(TT):
            xp_ref[s * bt:(s + 1) * bt, :] = xp3[:, s, :]

        len_m1 = len_ref[...] - 1
        h = h_ref[...]
        last = last_ref[...]
        w2 = w2_ref[...]
        for s in range(TT):
            p = jnp.dot(h, w2, preferred_element_type=jnp.float32)
            h = jnp.tanh(p + xp_ref[s * bt:(s + 1) * bt, :])
            g = tbase + s
            last = jnp.where(len_m1 == g - 1, h[:, H:], last)
        h_ref[...] = h
        last_ref[...] = last

    @pl.when(t == n_t - 1)
    def _final():
        len_m1 = len_ref[...] - 1
        pf = (jnp.dot(h_ref[...], w2_ref[...],
                      preferred_element_type=jnp.float32)[:, H:]
              + brow_ref[:, H:])
        h1f = jnp.tanh(pf)
        last = jnp.where(len_m1 == n_t * TT - 1, h1f, last_ref[...])
        logits = (jnp.dot(last, wfcp_ref[...],
                          preferred_element_type=jnp.float32) + bfcp_ref[...])
        m = jnp.max(logits, axis=1, keepdims=True)
        sh = logits - m
        lse = jnp.log(jnp.sum(jnp.exp(sh), axis=1, keepdims=True))
        out_ref[...] = sh - lse


@functools.partial(jax.jit, static_argnames=())
def kernel(x, lengths, wih0, whh0, b0, wih1, whh1, b1, wfc, bfc):
    B, T, D = x.shape
    B_pad = _round_up(max(B, 8), 8)
    bt = BT if B_pad % BT == 0 else B_pad
    n_b = B_pad // bt
    T_pad = _round_up(T, TT)
    n_t = T_pad // TT

    xf = x.astype(jnp.float32)
    if B_pad != B or T_pad != T:
        xf = jnp.pad(xf, ((0, B_pad - B), (0, T_pad - T), (0, 0)))

    len_pad = lengths.astype(jnp.int32)
    if B_pad != B:
        len_pad = jnp.pad(len_pad, (0, B_pad - B), constant_values=1)
    len_col = len_pad.reshape(B_pad, 1)

    out = pl.pallas_call(
        _rnn_kernel,
        out_shape=jax.ShapeDtypeStruct((B_pad, O_REAL), jnp.float32),
        grid=(n_b, n_t),
        in_specs=[
            pl.BlockSpec((bt, 1), lambda b, t: (b, 0)),
            pl.BlockSpec((bt, TT, D), lambda b, t: (b, t, 0)),
            pl.BlockSpec((H, D_IN), lambda b, t: (0, 0)),
            pl.BlockSpec((H, H), lambda b, t: (0, 0)),
            pl.BlockSpec((H, H), lambda b, t: (0, 0)),
            pl.BlockSpec((H, H), lambda b, t: (0, 0)),
            pl.BlockSpec((1, H), lambda b, t: (0, 0)),
            pl.BlockSpec((1, H), lambda b, t: (0, 0)),
            pl.BlockSpec((O_REAL, H), lambda b, t: (0, 0)),
            pl.BlockSpec((1, O_REAL), lambda b, t: (0, 0)),
        ],
        out_specs=pl.BlockSpec((bt, OP), lambda b, t: (b, 0)),
        scratch_shapes=[
            pltpu.VMEM((D_IN, HH), jnp.float32),
            pltpu.VMEM((HH, HH), jnp.float32),
            pltpu.VMEM((1, HH), jnp.float32),
            pltpu.VMEM((H, OP), jnp.float32),
            pltpu.VMEM((1, OP), jnp.float32),
            pltpu.VMEM((TT * bt, HH), jnp.float32),
            pltpu.VMEM((bt, HH), jnp.float32),
            pltpu.VMEM((bt, H), jnp.float32),
        ],
        compiler_params=pltpu.CompilerParams(
            dimension_semantics=("parallel", "arbitrary"),
            vmem_limit_bytes=60 * 2**20),
    )(len_col, xf, wih0, whh0, wih1, whh1,
      b0.reshape(1, H), b1.reshape(1, H), wfc, bfc.reshape(1, O_REAL))

    return out if B_pad == B else out[:B]


# speedup vs baseline: 1.1174x; 1.0003x over previous
import functools

import jax
import jax.numpy as jnp
from jax.experimental import pallas as pl
from jax.experimental.pallas import tpu as pltpu

D_IN = 768
H = 128
HH = 2 * H
O_REAL = 85
OP = 128
NEG_BIG = -1e30
TT = 16
BT = 256


def _round_up(x, m):
    return (x + m - 1) // m * m


def _rnn_kernel(len_ref, x_ref, xb_ref, wih0_ref, whh0_ref, wih1_ref, whh1_ref,
                b0_ref, b1_ref, wfc_ref, bfc_ref, out_ref,
                w1_ref, w2_ref, brow_ref, wfcp_ref, bfcp_ref,
                xp_ref, h_ref, last_ref):
    t = pl.program_id(1)
    n_t = pl.num_programs(1)
    tbase = t * TT
    bt = h_ref.shape[0]

    @pl.when(t == 0)
    def _init():
        h_ref[...] = jnp.zeros_like(h_ref)
        last_ref[...] = jnp.zeros_like(last_ref)
        w1_ref[:, :H] = jnp.swapaxes(wih0_ref[...], 0, 1)
        w1_ref[:, H:] = jnp.zeros((D_IN, H), jnp.float32)
        w2_ref[:H, :H] = jnp.swapaxes(whh0_ref[...], 0, 1)
        w2_ref[:H, H:] = jnp.swapaxes(wih1_ref[...], 0, 1)
        w2_ref[H:, :H] = jnp.zeros((H, H), jnp.float32)
        w2_ref[H:, H:] = jnp.swapaxes(whh1_ref[...], 0, 1)
        brow_ref[:, :H] = b0_ref[...]
        brow_ref[:, H:] = b1_ref[...]
        wfcp_ref[...] = jnp.concatenate(
            [jnp.swapaxes(wfc_ref[...], 0, 1),
             jnp.zeros((H, OP - O_REAL), jnp.float32)], axis=1)
        bfcp_ref[...] = jnp.concatenate(
            [bfc_ref[...],
             jnp.full((1, OP - O_REAL), NEG_BIG, jnp.float32)], axis=1)

    @pl.when(tbase <= jnp.max(len_ref[...]))
    def _compute():
        hb = bt // 2
        for k, xr in ((0, x_ref), (1, xb_ref)):
            xv = xr[...].reshape(hb * TT, D_IN)
            xp_all = (jnp.dot(xv, w1_ref[...],
                              preferred_element_type=jnp.float32)
                      + brow_ref[...])
            xp3 = xp_all.reshape(hb, TT, HH)
            for s in range(TT):
                r = s * bt + k * hb
                xp_ref[r:r + hb, :] = xp3[:, s, :]

        len_m1 = len_ref[...] - 1
        h = h_ref[...]
        last = last_ref[...]
        w2 = w2_ref[...]
        for s in range(TT):
            p = jnp.dot(h, w2, preferred_element_type=jnp.float32)
            h = jnp.tanh(p + xp_ref[s * bt:(s + 1) * bt, :])
            g = tbase + s
            last = jnp.where(len_m1 == g - 1, h[:, H:], last)
        h_ref[...] = h
        last_ref[...] = last

    @pl.when(t == n_t - 1)
    def _final():
        len_m1 = len_ref[...] - 1
        pf = (jnp.dot(h_ref[...], w2_ref[...],
                      preferred_element_type=jnp.float32)[:, H:]
              + brow_ref[:, H:])
        h1f = jnp.tanh(pf)
        last = jnp.where(len_m1 == n_t * TT - 1, h1f, last_ref[...])
        logits = (jnp.dot(last, wfcp_ref[...],
                          preferred_element_type=jnp.float32) + bfcp_ref[...])
        m = jnp.max(logits, axis=1, keepdims=True)
        sh = logits - m
        lse = jnp.log(jnp.sum(jnp.exp(sh), axis=1, keepdims=True))
        out_ref[...] = sh - lse


@functools.partial(jax.jit, static_argnames=())
def kernel(x, lengths, wih0, whh0, b0, wih1, whh1, b1, wfc, bfc):
    B, T, D = x.shape
    B_pad = _round_up(max(B, 8), 8)
    bt = BT if B_pad % BT == 0 else B_pad
    n_b = B_pad // bt
    T_pad = _round_up(T, TT)
    n_t = T_pad // TT

    xf = x.astype(jnp.float32)
    if B_pad != B or T_pad != T:
        xf = jnp.pad(xf, ((0, B_pad - B), (0, T_pad - T), (0, 0)))

    len_pad = lengths.astype(jnp.int32)
    if B_pad != B:
        len_pad = jnp.pad(len_pad, (0, B_pad - B), constant_values=1)
    len_col = len_pad.reshape(B_pad, 1)

    out = pl.pallas_call(
        _rnn_kernel,
        out_shape=jax.ShapeDtypeStruct((B_pad, O_REAL), jnp.float32),
        grid=(n_b, n_t),
        in_specs=[
            pl.BlockSpec((bt, 1), lambda b, t: (b, 0)),
            pl.BlockSpec((bt // 2, TT, D), lambda b, t: (2 * b, t, 0)),
            pl.BlockSpec((bt // 2, TT, D), lambda b, t: (2 * b + 1, t, 0)),
            pl.BlockSpec((H, D_IN), lambda b, t: (0, 0)),
            pl.BlockSpec((H, H), lambda b, t: (0, 0)),
            pl.BlockSpec((H, H), lambda b, t: (0, 0)),
            pl.BlockSpec((H, H), lambda b, t: (0, 0)),
            pl.BlockSpec((1, H), lambda b, t: (0, 0)),
            pl.BlockSpec((1, H), lambda b, t: (0, 0)),
            pl.BlockSpec((O_REAL, H), lambda b, t: (0, 0)),
            pl.BlockSpec((1, O_REAL), lambda b, t: (0, 0)),
        ],
        out_specs=pl.BlockSpec((bt, OP), lambda b, t: (b, 0)),
        scratch_shapes=[
            pltpu.VMEM((D_IN, HH), jnp.float32),
            pltpu.VMEM((HH, HH), jnp.float32),
            pltpu.VMEM((1, HH), jnp.float32),
            pltpu.VMEM((H, OP), jnp.float32),
            pltpu.VMEM((1, OP), jnp.float32),
            pltpu.VMEM((TT * bt, HH), jnp.float32),
            pltpu.VMEM((bt, HH), jnp.float32),
            pltpu.VMEM((bt, H), jnp.float32),
        ],
        compiler_params=pltpu.CompilerParams(
            dimension_semantics=("parallel", "arbitrary"),
            vmem_limit_bytes=60 * 2**20),
    )(len_col, xf, xf, wih0, whh0, wih1, whh1,
      b0.reshape(1, H), b1.reshape(1, H), wfc, bfc.reshape(1, O_REAL))

    return out if B_pad == B else out[:B]


# speedup vs baseline: 1.1203x; 1.0026x over previous
import functools

import jax
import jax.numpy as jnp
from jax.experimental import pallas as pl
from jax.experimental.pallas import tpu as pltpu

D_IN = 768
H = 128
HH = 2 * H
O_REAL = 85
OP = 128
NEG_BIG = -1e30
TT = 16
BT = 256


def _round_up(x, m):
    return (x + m - 1) // m * m


def _rnn_kernel(len_ref, x_ref, wih0_ref, whh0_ref, wih1_ref, whh1_ref,
                b0_ref, b1_ref, wfc_ref, bfc_ref, out_ref,
                w1_ref, w2_ref, brow_ref, wfcp_ref, bfcp_ref,
                xp_ref, h_ref, last_ref):
    t = pl.program_id(1)
    n_t = pl.num_programs(1)
    tbase = t * TT
    bt = h_ref.shape[0]

    @pl.when(t == 0)
    def _init():
        h_ref[...] = jnp.zeros_like(h_ref)
        last_ref[...] = jnp.zeros_like(last_ref)
        w1_ref[:, :H] = jnp.swapaxes(wih0_ref[...], 0, 1)
        w1_ref[:, H:] = jnp.zeros((D_IN, H), jnp.float32)
        w2_ref[:H, :H] = jnp.swapaxes(whh0_ref[...], 0, 1)
        w2_ref[:H, H:] = jnp.swapaxes(wih1_ref[...], 0, 1)
        w2_ref[H:, :H] = jnp.zeros((H, H), jnp.float32)
        w2_ref[H:, H:] = jnp.swapaxes(whh1_ref[...], 0, 1)
        brow_ref[:, :H] = b0_ref[...]
        brow_ref[:, H:] = b1_ref[...]
        wfcp_ref[...] = jnp.concatenate(
            [jnp.swapaxes(wfc_ref[...], 0, 1),
             jnp.zeros((H, OP - O_REAL), jnp.float32)], axis=1)
        bfcp_ref[...] = jnp.concatenate(
            [bfc_ref[...],
             jnp.full((1, OP - O_REAL), NEG_BIG, jnp.float32)], axis=1)

    @pl.when(tbase <= jnp.max(len_ref[...]))
    def _compute():
        xv = x_ref[...].reshape(bt * TT, D_IN)
        xp_all = (jnp.dot(xv, w1_ref[...],
                          preferred_element_type=jnp.float32)
                  + brow_ref[...])
        xp3 = xp_all.reshape(bt, TT, HH)
        for s in range(TT):
            xp_ref[s * bt:(s + 1) * bt, :] = xp3[:, s, :]

        len_m1 = len_ref[...] - 1
        h = h_ref[...]
        last = last_ref[...]
        w2 = w2_ref[...]
        for s in range(TT):
            p = jnp.dot(h, w2, preferred_element_type=jnp.float32)
            h = jnp.tanh(p + xp_ref[s * bt:(s + 1) * bt, :])
            g = tbase + s
            last = jnp.where(len_m1 == g - 1, h[:, H:], last)
        h_ref[...] = h
        last_ref[...] = last

    @pl.when(t == n_t - 1)
    def _final():
        len_m1 = len_ref[...] - 1
        pf = (jnp.dot(h_ref[...], w2_ref[...],
                      preferred_element_type=jnp.float32)[:, H:]
              + brow_ref[:, H:])
        h1f = jnp.tanh(pf)
        last = jnp.where(len_m1 == n_t * TT - 1, h1f, last_ref[...])
        logits = (jnp.dot(last, wfcp_ref[...],
                          preferred_element_type=jnp.float32) + bfcp_ref[...])
        m = jnp.max(logits, axis=1, keepdims=True)
        sh = logits - m
        lse = jnp.log(jnp.sum(jnp.exp(sh), axis=1, keepdims=True))
        out_ref[...] = sh - lse


@functools.partial(jax.jit, static_argnames=())
def kernel(x, lengths, wih0, whh0, b0, wih1, whh1, b1, wfc, bfc):
    B, T, D = x.shape
    B_pad = _round_up(max(B, 8), 8)
    bt = BT if B_pad % BT == 0 else B_pad
    n_b = B_pad // bt
    T_pad = _round_up(T, TT)
    n_t = T_pad // TT

    xf = x.astype(jnp.float32)
    if B_pad != B or T_pad != T:
        xf = jnp.pad(xf, ((0, B_pad - B), (0, T_pad - T), (0, 0)))

    len_pad = lengths.astype(jnp.int32)
    if B_pad != B:
        len_pad = jnp.pad(len_pad, (0, B_pad - B), constant_values=1)
    len_col = len_pad.reshape(B_pad, 1)

    out = pl.pallas_call(
        _rnn_kernel,
        out_shape=jax.ShapeDtypeStruct((B_pad, O_REAL), jnp.float32),
        grid=(n_b, n_t),
        in_specs=[
            pl.BlockSpec((bt, 1), lambda b, t: (b, 0)),
            pl.BlockSpec((bt, TT, D), lambda b, t: (b, t, 0)),
            pl.BlockSpec((H, D_IN), lambda b, t: (0, 0)),
            pl.BlockSpec((H, H), lambda b, t: (0, 0)),
            pl.BlockSpec((H, H), lambda b, t: (0, 0)),
            pl.BlockSpec((H, H), lambda b, t: (0, 0)),
            pl.BlockSpec((1, H), lambda b, t: (0, 0)),
            pl.BlockSpec((1, H), lambda b, t: (0, 0)),
            pl.BlockSpec((O_REAL, H), lambda b, t: (0, 0)),
            pl.BlockSpec((1, O_REAL), lambda b, t: (0, 0)),
        ],
        out_specs=pl.BlockSpec((bt, OP), lambda b, t: (b, 0)),
        scratch_shapes=[
            pltpu.VMEM((D_IN, HH), jnp.float32),
            pltpu.VMEM((HH, HH), jnp.float32),
            pltpu.VMEM((1, HH), jnp.float32),
            pltpu.VMEM((H, OP), jnp.float32),
            pltpu.VMEM((1, OP), jnp.float32),
            pltpu.VMEM((TT * bt, HH), jnp.float32),
            pltpu.VMEM((bt, HH), jnp.float32),
            pltpu.VMEM((bt, H), jnp.float32),
        ],
        compiler_params=pltpu.CompilerParams(
            dimension_semantics=("parallel", "arbitrary"),
            vmem_limit_bytes=60 * 2**20),
    )(len_col, xf, wih0, whh0, wih1, whh1,
      b0.reshape(1, H), b1.reshape(1, H), wfc, bfc.reshape(1, O_REAL))

    return out if B_pad == B else out[:B]


# speedup vs baseline: 1.1222x; 1.0016x over previous
import functools

import jax
import jax.numpy as jnp
from jax.experimental import pallas as pl
from jax.experimental.pallas import tpu as pltpu

D_IN = 768
H = 128
HH = 2 * H
O_REAL = 85
OP = 128
NEG_BIG = -1e30
TT = 16
BT = 256


def _round_up(x, m):
    return (x + m - 1) // m * m


def _rnn_kernel(len_ref, x_ref, wih0_ref, whh0_ref, wih1_ref, whh1_ref,
                b0_ref, b1_ref, wfc_ref, bfc_ref, out_ref,
                w1_ref, w2_ref, brow_ref, wfcp_ref, bfcp_ref,
                xp_ref, h_ref, last_ref):
    t = pl.program_id(1)
    n_t = pl.num_programs(1)
    tbase = t * TT
    bt = h_ref.shape[0]

    @pl.when(t == 0)
    def _init():
        h_ref[...] = jnp.zeros_like(h_ref)
        last_ref[...] = jnp.zeros_like(last_ref)
        w1_ref[:, :H] = jnp.swapaxes(wih0_ref[...], 0, 1)
        w1_ref[:, H:] = jnp.zeros((D_IN, H), jnp.float32)
        w2_ref[:H, :H] = jnp.swapaxes(whh0_ref[...], 0, 1)
        w2_ref[:H, H:] = jnp.swapaxes(wih1_ref[...], 0, 1)
        w2_ref[H:, :H] = jnp.zeros((H, H), jnp.float32)
        w2_ref[H:, H:] = jnp.swapaxes(whh1_ref[...], 0, 1)
        brow_ref[:, :H] = b0_ref[...]
        brow_ref[:, H:] = b1_ref[...]
        wfcp_ref[...] = jnp.concatenate(
            [jnp.swapaxes(wfc_ref[...], 0, 1),
             jnp.zeros((H, OP - O_REAL), jnp.float32)], axis=1)
        bfcp_ref[...] = jnp.concatenate(
            [bfc_ref[...],
             jnp.full((1, OP - O_REAL), NEG_BIG, jnp.float32)], axis=1)

    @pl.when(tbase <= jnp.max(len_ref[...]))
    def _compute():
        xv = x_ref[...].reshape(bt * TT, D_IN)
        xp_all = (jnp.dot(xv, w1_ref[...],
                          preferred_element_type=jnp.float32)
                  + brow_ref[...])
        xp3 = xp_all.reshape(bt, TT, HH)
        for s in range(TT):
            xp_ref[s * bt:(s + 1) * bt, :] = xp3[:, s, :]

        len_m1 = len_ref[...] - 1
        h = h_ref[...]
        last = last_ref[...]
        w2 = w2_ref[...]
        for s in range(TT):
            p = jnp.dot(h, w2, preferred_element_type=jnp.float32)
            h = jnp.tanh(p + xp_ref[s * bt:(s + 1) * bt, :])
            if s == 0:
                lane = jax.lax.broadcasted_iota(jnp.int32, (1, HH), 1)
                h = jnp.where((t > 0) | (lane < H), h, 0.0)
            g = tbase + s
            last = jnp.where(len_m1 == g - 1, h[:, H:], last)
        h_ref[...] = h
        last_ref[...] = last

    @pl.when(t == n_t - 1)
    def _final():
        len_m1 = len_ref[...] - 1
        pf = (jnp.dot(h_ref[...], w2_ref[...],
                      preferred_element_type=jnp.float32)[:, H:]
              + brow_ref[:, H:])
        h1f = jnp.tanh(pf)
        last = jnp.where(len_m1 == n_t * TT - 1, h1f, last_ref[...])
        logits = (jnp.dot(last, wfcp_ref[...],
                          preferred_element_type=jnp.float32) + bfcp_ref[...])
        m = jnp.max(logits, axis=1, keepdims=True)
        sh = logits - m
        lse = jnp.log(jnp.sum(jnp.exp(sh), axis=1, keepdims=True))
        out_ref[...] = sh - lse


@functools.partial(jax.jit, static_argnames=())
def kernel(x, lengths, wih0, whh0, b0, wih1, whh1, b1, wfc, bfc):
    B, T, D = x.shape
    B_pad = _round_up(max(B, 8), 8)
    bt = BT if B_pad % BT == 0 else B_pad
    n_b = B_pad // bt
    T_pad = _round_up(T, TT)
    n_t = T_pad // TT

    xf = x.astype(jnp.float32)
    if B_pad != B or T_pad != T:
        xf = jnp.pad(xf, ((0, B_pad - B), (0, T_pad - T), (0, 0)))

    len_pad = lengths.astype(jnp.int32)
    if B_pad != B:
        len_pad = jnp.pad(len_pad, (0, B_pad - B), constant_values=1)
    len_col = len_pad.reshape(B_pad, 1)

    out = pl.pallas_call(
        _rnn_kernel,
        out_shape=jax.ShapeDtypeStruct((B_pad, O_REAL), jnp.float32),
        grid=(n_b, n_t),
        in_specs=[
            pl.BlockSpec((bt, 1), lambda b, t: (b, 0)),
            pl.BlockSpec((bt, TT, D), lambda b, t: (b, t, 0)),
            pl.BlockSpec((H, D_IN), lambda b, t: (0, 0)),
            pl.BlockSpec((H, H), lambda b, t: (0, 0)),
            pl.BlockSpec((H, H), lambda b, t: (0, 0)),
            pl.BlockSpec((H, H), lambda b, t: (0, 0)),
            pl.BlockSpec((1, H), lambda b, t: (0, 0)),
            pl.BlockSpec((1, H), lambda b, t: (0, 0)),
            pl.BlockSpec((O_REAL, H), lambda b, t: (0, 0)),
            pl.BlockSpec((1, O_REAL), lambda b, t: (0, 0)),
        ],
        out_specs=pl.BlockSpec((bt, OP), lambda b, t: (b, 0)),
        scratch_shapes=[
            pltpu.VMEM((D_IN, HH), jnp.float32),
            pltpu.VMEM((HH, HH), jnp.float32),
            pltpu.VMEM((1, HH), jnp.float32),
            pltpu.VMEM((H, OP), jnp.float32),
            pltpu.VMEM((1, OP), jnp.float32),
            pltpu.VMEM((TT * bt, HH), jnp.float32),
            pltpu.VMEM((bt, HH), jnp.float32),
            pltpu.VMEM((bt, H), jnp.float32),
        ],
        compiler_params=pltpu.CompilerParams(
            dimension_semantics=("parallel", "arbitrary"),
            vmem_limit_bytes=60 * 2**20),
    )(len_col, xf, wih0, whh0, wih1, whh1,
      b0.reshape(1, H), b1.reshape(1, H), wfc, bfc.reshape(1, O_REAL))

    return out if B_pad == B else out[:B]
